# Initial kernel scaffold; baseline (speedup 1.0000x reference)
import jax
import jax.numpy as jnp
from jax.experimental import pallas as pl


def kernel(image_feat, pos_embed, params, gumbel):
    raise NotImplementedError("write your pallas kernel here")


# speedup vs baseline: 2.1548x; 2.1548x over previous
import functools

import jax
import jax.numpy as jnp
from jax.experimental import pallas as pl
from jax.experimental.pallas import tpu as pltpu

_HEADS = 8
_DH = 48
_L = 2048
_D = 384
_VIS = 204
_VIS_PAD = 256
_RB = 512


def _cumsum_lanes(x):
    n = x.shape[1]
    sh = 1
    while sh < n:
        x = x + jnp.concatenate(
            [jnp.zeros((1, sh), x.dtype), x[:, :n - sh]], axis=1)
        sh *= 2
    return x


def _layernorm(x, w, b, eps=1e-5):
    mu = jnp.mean(x, axis=-1, keepdims=True)
    var = jnp.var(x, axis=-1, keepdims=True)
    return (x - mu) / jnp.sqrt(var + eps) * w + b


def _dot(a, b):
    return jax.lax.dot_general(a, b, (((1,), (0,)), ((), ())),
                               precision=jax.lax.Precision.DEFAULT,
                               preferred_element_type=jnp.float32)


def _dot_nt(a, b):
    return jax.lax.dot_general(a, b, (((1,), (1,)), ((), ())),
                               precision=jax.lax.Precision.DEFAULT,
                               preferred_element_type=jnp.float32)


def _mega_kernel(x_ref, pep_ref, pe_ref, gum_ref,
                 wq_ref, wk_ref, wv_ref, projt_ref, projb_ref,
                 ln1w_ref, ln1b_ref, ln2w_ref, ln2b_ref,
                 fc1t_ref, fc1b_ref, fc2t_ref, fc2b_ref,
                 headt_ref, headb_ref, normw_ref, normb_ref,
                 prob_ref, vis_ref, mask_ref):
    x = x_ref[0]
    h0 = x + pep_ref[0]
    h = _layernorm(h0, ln1w_ref[...], ln1b_ref[...])

    q = _dot(h, wq_ref[...])
    k = _dot(h, wk_ref[...])
    v = _dot(h, wv_ref[...])

    scale = _DH ** -0.5
    o_heads = []
    for hd in range(_HEADS):
        sl = slice(hd * _DH, (hd + 1) * _DH)
        q_h, k_h, v_h = q[:, sl], k[:, sl], v[:, sl]
        o_blocks = []
        for rb in range(_L // _RB):
            qs = q_h[rb * _RB:(rb + 1) * _RB, :]
            s = _dot_nt(qs, k_h) * scale
            m = jnp.max(s, axis=1, keepdims=True)
            p = jnp.exp(s - m)
            denom = jnp.sum(p, axis=1, keepdims=True)
            a = p / denom
            o_blocks.append(_dot(a, v_h))
        o_heads.append(jnp.concatenate(o_blocks, axis=0))
    o = jnp.concatenate(o_heads, axis=1)

    x2 = h0 + (_dot(o, projt_ref[...]) + projb_ref[...])

    x3_blocks = []
    for rb in range(_L // _RB):
        x2b = x2[rb * _RB:(rb + 1) * _RB, :]
        h2 = _layernorm(x2b, ln2w_ref[...], ln2b_ref[...])
        pre = _dot(h2, fc1t_ref[...]) + fc1b_ref[...]
        g = pre * 0.5 * (1.0 + jax.lax.erf(pre * (2.0 ** -0.5)))
        x3_blocks.append(x2b + (_dot(g, fc2t_ref[...]) + fc2b_ref[...]))
    x3 = jnp.concatenate(x3_blocks, axis=0)

    logit_col = _dot(x3, headt_ref[...]) + headb_ref[...]
    logits = jnp.nan_to_num(jnp.transpose(logit_col, (1, 0)))

    lm = jnp.max(logits, axis=1, keepdims=True)
    le = jnp.exp(logits - lm)
    prob = le / jnp.sum(le, axis=1, keepdims=True)
    prob_ref[0] = prob

    scores = jnp.log(prob + 1e-20) + gum_ref[0]

    bits = jax.lax.bitcast_convert_type(scores, jnp.int32)
    keys = jnp.where(bits >= 0, bits, bits ^ jnp.int32(0x7FFFFFFF))

    def bs_body(_, carry):
        lo, hi = carry
        mid = (lo >> 1) + (hi >> 1) + (lo & hi & 1)
        cnt = jnp.sum(jnp.where(keys > mid, 1.0, 0.0))
        big = cnt >= float(_VIS)
        lo = jnp.where(big, mid + 1, lo)
        hi = jnp.where(big, hi, mid)
        return lo, hi

    lo0 = jnp.full((1, 1), -2147483648, jnp.int32)
    hi0 = jnp.full((1, 1), 2147483647, jnp.int32)
    lo, hi = jax.lax.fori_loop(0, 32, bs_body, (lo0, hi0))
    tstar = lo

    strict = keys > tstar
    eq = keys == tstar
    n_strict = jnp.sum(jnp.where(strict, 1.0, 0.0))
    tie_rank = _cumsum_lanes(jnp.where(eq, 1.0, 0.0))
    sel = strict | (eq & (tie_rank <= (float(_VIS) - n_strict)))
    sel_f = jnp.where(sel, 1.0, 0.0)
    mask_ref[0] = 1.0 - sel_f

    rank = (_cumsum_lanes(sel_f) - 1.0).astype(jnp.int32)
    row_iota = jax.lax.broadcasted_iota(jnp.int32, (_VIS_PAD, _L), 0)
    onehot = jnp.where((row_iota == rank) & sel, 1.0, 0.0)
    xf = x + pe_ref[0]
    vis = _dot(onehot, xf)
    vis_ref[0] = _layernorm(vis, normw_ref[...], normb_ref[...])


@jax.jit
def kernel(image_feat, pos_embed, params, gumbel):
    B = image_feat.shape[0]
    x = image_feat.reshape(B, _L, _D)
    p = params
    row = lambda a: a.reshape(1, -1)

    operands = (
        x, p['pos_embed_probs'], pos_embed, gumbel.reshape(B, 1, _L),
        p['qkv_w'][0 * _D:1 * _D].T, p['qkv_w'][1 * _D:2 * _D].T,
        p['qkv_w'][2 * _D:3 * _D].T, p['proj_w'].T, row(p['proj_b']),
        row(p['norm1_w']), row(p['norm1_b']), row(p['norm2_w']), row(p['norm2_b']),
        p['fc1_w'].T, row(p['fc1_b']), p['fc2_w'].T, row(p['fc2_b']),
        p['head_w'].T, row(p['head_b']), row(p['norm_w']), row(p['norm_b']),
    )

    in_specs = [
        pl.BlockSpec((1, _L, _D), lambda b: (b, 0, 0)),
        pl.BlockSpec((1, _L, _D), lambda b: (0, 0, 0)),
        pl.BlockSpec((1, _L, _D), lambda b: (0, 0, 0)),
        pl.BlockSpec((1, 1, _L), lambda b: (b, 0, 0)),
    ] + [pl.BlockSpec(a.shape, functools.partial(lambda n, b: (0,) * n, a.ndim))
         for a in operands[4:]]

    out_shape = (
        jax.ShapeDtypeStruct((B, 1, _L), jnp.float32),
        jax.ShapeDtypeStruct((B, _VIS_PAD, _D), jnp.float32),
        jax.ShapeDtypeStruct((B, 1, _L), jnp.float32),
    )
    out_specs = (
        pl.BlockSpec((1, 1, _L), lambda b: (b, 0, 0)),
        pl.BlockSpec((1, _VIS_PAD, _D), lambda b: (b, 0, 0)),
        pl.BlockSpec((1, 1, _L), lambda b: (b, 0, 0)),
    )

    prob, vis_pad, mask_f = pl.pallas_call(
        _mega_kernel,
        grid=(B,),
        in_specs=in_specs,
        out_specs=out_specs,
        out_shape=out_shape,
    )(*operands)

    return (prob.reshape(B, _L), vis_pad[:, :_VIS, :],
            mask_f.reshape(B, _L).astype(bool))
